# baseline (speedup 1.0000x reference)
import functools

import jax
import jax.numpy as jnp
from jax import lax
from jax.experimental import pallas as pl
from jax.experimental.pallas import tpu as pltpu
from jax.experimental.pallas import tpu_sc as plsc

N_NODES = 100000
N_EDGES = 1600000
NODE_FEAT = 128
NODE_EMB = 128
EDGE_EMB = 16
N_EDGE_TYPE = 16

NC = 2
NS = 16
NW = NC * NS

EDGES_PER_W = N_EDGES // NW
CHUNK = 5000
N_CHUNKS = EDGES_PER_W // CHUNK

_SC_MESH = plsc.VectorSubcoreMesh(
    core_axis_name="c", subcore_axis_name="s", num_cores=NC, num_subcores=NS
)


@functools.partial(
    pl.kernel,
    out_type=jax.ShapeDtypeStruct((N_EDGE_TYPE, EDGE_EMB), jnp.float32),
    mesh=_SC_MESH,
    scratch_types=[
        pltpu.VMEM((N_EDGE_TYPE, EDGE_EMB), jnp.float32),
        pltpu.VMEM((EDGE_EMB, EDGE_EMB), jnp.float32),
        pltpu.VMEM((EDGE_EMB,), jnp.float32),
        pltpu.VMEM((N_EDGE_TYPE, EDGE_EMB), jnp.float32),
    ],
)
def _fused_table_sc(emb_hbm, w2t_hbm, b2_hbm, out_hbm, emb_v, w2t_v, b2_v, out_v):
    wid = lax.axis_index("s") * NC + lax.axis_index("c")

    @pl.when(wid == 0)
    def _():
        pltpu.sync_copy(emb_hbm, emb_v)
        pltpu.sync_copy(w2t_hbm, w2t_v)
        pltpu.sync_copy(b2_hbm, b2_v)
        for i in range(N_EDGE_TYPE):
            e_row = emb_v[i, :]
            acc = b2_v[...]
            for k in range(EDGE_EMB):
                acc = acc + e_row[k] * w2t_v[k, :]
            out_v[i, :] = acc
        pltpu.sync_copy(out_v, out_hbm)


@functools.partial(
    pl.kernel,
    out_type=jax.ShapeDtypeStruct((N_EDGES, EDGE_EMB), jnp.float32),
    mesh=_SC_MESH,
    scratch_types=[
        pltpu.VMEM((CHUNK,), jnp.int32),
        pltpu.VMEM((CHUNK, EDGE_EMB), jnp.float32),
        pltpu.SemaphoreType.DMA,
    ],
    compiler_params=pltpu.CompilerParams(use_tc_tiling_on_sc=False),
)
def _edge_gather_sc(idx_hbm, table_hbm, out_hbm, idx_v, rows_v, sem):
    wid = lax.axis_index("s") * NC + lax.axis_index("c")
    base_w = wid * EDGES_PER_W
    for c in range(N_CHUNKS):
        base = base_w + c * CHUNK
        pltpu.sync_copy(idx_hbm.at[pl.ds(base, CHUNK)], idx_v)
        pltpu.async_copy(table_hbm.at[idx_v], rows_v, sem).wait()
        pltpu.sync_copy(rows_v, out_hbm.at[pl.ds(base, CHUNK)])


NODE_BLK = 2000


def _node_mlp_body(x_ref, w_ref, b_ref, o_ref):
    o_ref[...] = (
        jnp.dot(x_ref[...], w_ref[...], preferred_element_type=jnp.float32)
        + b_ref[...]
    )


def _node_mlp(x, w1t, b1):
    grid = N_NODES // NODE_BLK
    return pl.pallas_call(
        _node_mlp_body,
        grid=(grid,),
        in_specs=[
            pl.BlockSpec((NODE_BLK, NODE_FEAT), lambda i: (i, 0)),
            pl.BlockSpec((NODE_FEAT, NODE_EMB), lambda i: (0, 0)),
            pl.BlockSpec((1, NODE_EMB), lambda i: (0, 0)),
        ],
        out_specs=pl.BlockSpec((NODE_BLK, NODE_EMB), lambda i: (i, 0)),
        out_shape=jax.ShapeDtypeStruct((N_NODES, NODE_EMB), jnp.float32),
    )(x, w1t, b1)


def kernel(raw_node_features, raw_edge_features, W1, b1, emb_table, W2, b2):
    fused = _fused_table_sc(emb_table, W2.T, b2)
    edge_outputs = _edge_gather_sc(raw_edge_features, fused)
    node_outputs = _node_mlp(raw_node_features, W1.T, b1.reshape(1, NODE_EMB))
    return (node_outputs, edge_outputs)


# speedup vs baseline: 3.8755x; 3.8755x over previous
import functools

import jax
import jax.numpy as jnp
from jax import lax
from jax.experimental import pallas as pl
from jax.experimental.pallas import tpu as pltpu
from jax.experimental.pallas import tpu_sc as plsc

N_NODES = 100000
N_EDGES = 1600000
NODE_FEAT = 128
NODE_EMB = 128
EDGE_EMB = 16
N_EDGE_TYPE = 16

NC = 2
NS = 16
NW = NC * NS
LANES = 16

EDGES_PER_W = N_EDGES // NW
CHUNK = 2000
N_CHUNKS = EDGES_PER_W // CHUNK

_SC_MESH = plsc.VectorSubcoreMesh(
    core_axis_name="c", subcore_axis_name="s", num_cores=NC, num_subcores=NS
)


@functools.partial(
    pl.kernel,
    out_type=jax.ShapeDtypeStruct((N_EDGES, EDGE_EMB), jnp.float32),
    mesh=_SC_MESH,
    scratch_types=[
        pltpu.VMEM((N_EDGE_TYPE, EDGE_EMB), jnp.float32),
        pltpu.VMEM((EDGE_EMB, EDGE_EMB), jnp.float32),
        pltpu.VMEM((EDGE_EMB,), jnp.float32),
        pltpu.VMEM((N_EDGE_TYPE, EDGE_EMB), jnp.float32),
        pltpu.VMEM((EDGES_PER_W,), jnp.int32),
        pltpu.VMEM((CHUNK, EDGE_EMB), jnp.float32),
        pltpu.VMEM((CHUNK, EDGE_EMB), jnp.float32),
        pltpu.SemaphoreType.DMA,
        pltpu.SemaphoreType.DMA,
        pltpu.SemaphoreType.DMA,
    ],
    compiler_params=pltpu.CompilerParams(
        use_tc_tiling_on_sc=False, needs_layout_passes=False
    ),
)
def _edge_path_sc(
    idx_hbm, emb_hbm, w2t_hbm, b2_hbm, out_hbm,
    emb_v, w2t_v, b2_v, table_v, idx_v, rows0, rows1, sem_idx, sem_o0, sem_o1,
):
    wid = lax.axis_index("s") * NC + lax.axis_index("c")
    base_w = wid * EDGES_PER_W

    idx_dma = pltpu.async_copy(
        idx_hbm.at[pl.ds(base_w, EDGES_PER_W)], idx_v, sem_idx
    )

    pltpu.sync_copy(emb_hbm, emb_v)
    pltpu.sync_copy(w2t_hbm, w2t_v)
    pltpu.sync_copy(b2_hbm, b2_v)
    for i in range(N_EDGE_TYPE):
        e_row = emb_v[i, :]
        acc = b2_v[...]
        for k in range(EDGE_EMB):
            acc = acc + e_row[k] * w2t_v[k, :]
        table_v[i, :] = acc

    idx_dma.wait()

    iota16 = lax.iota(jnp.int32, LANES)
    cols = [jnp.full((LANES,), j, dtype=jnp.int32) for j in range(EDGE_EMB)]
    rows_bufs = (rows0, rows1)
    sems = (sem_o0, sem_o1)
    out_desc = {}

    for c in range(N_CHUNKS):
        b = c & 1
        if c >= 2:
            out_desc[c - 2].wait()
        rb = rows_bufs[b]
        ib_base = c * CHUNK

        def g_body(g, _, rb=rb, ib_base=ib_base):
            idxv = idx_v[pl.ds(ib_base + g * LANES, LANES)]
            row_pos = g * LANES + iota16
            for j in range(EDGE_EMB):
                vals = plsc.load_gather(table_v, [idxv, cols[j]])
                plsc.store_scatter(rb, [row_pos, cols[j]], vals)
            return _

        lax.fori_loop(0, CHUNK // LANES, g_body, 0)
        out_desc[c] = pltpu.async_copy(
            rb, out_hbm.at[pl.ds(base_w + c * CHUNK, CHUNK)], sems[b]
        )

    out_desc[N_CHUNKS - 2].wait()
    out_desc[N_CHUNKS - 1].wait()


NODE_BLK = 2000


def _node_mlp_body(x_ref, w_ref, b_ref, o_ref):
    o_ref[...] = (
        jnp.dot(x_ref[...], w_ref[...], preferred_element_type=jnp.float32)
        + b_ref[...]
    )


def _node_mlp(x, w1t, b1):
    grid = N_NODES // NODE_BLK
    return pl.pallas_call(
        _node_mlp_body,
        grid=(grid,),
        in_specs=[
            pl.BlockSpec((NODE_BLK, NODE_FEAT), lambda i: (i, 0)),
            pl.BlockSpec((NODE_FEAT, NODE_EMB), lambda i: (0, 0)),
            pl.BlockSpec((1, NODE_EMB), lambda i: (0, 0)),
        ],
        out_specs=pl.BlockSpec((NODE_BLK, NODE_EMB), lambda i: (i, 0)),
        out_shape=jax.ShapeDtypeStruct((N_NODES, NODE_EMB), jnp.float32),
    )(x, w1t, b1)


def kernel(raw_node_features, raw_edge_features, W1, b1, emb_table, W2, b2):
    edge_outputs = _edge_path_sc(raw_edge_features, emb_table, W2.T, b2)
    node_outputs = _node_mlp(raw_node_features, W1.T, b1.reshape(1, NODE_EMB))
    return (node_outputs, edge_outputs)


# speedup vs baseline: 4.4982x; 1.1607x over previous
import functools

import jax
import jax.numpy as jnp
from jax import lax
from jax.experimental import pallas as pl
from jax.experimental.pallas import tpu as pltpu
from jax.experimental.pallas import tpu_sc as plsc

N_NODES = 100000
N_EDGES = 1600000
NODE_FEAT = 128
NODE_EMB = 128
EDGE_EMB = 16
N_EDGE_TYPE = 16

NC = 2
NS = 16
NW = NC * NS
LANES = 16

EDGES_PER_W = N_EDGES // NW
CHUNK = 2000
N_CHUNKS = EDGES_PER_W // CHUNK

_SC_MESH = plsc.VectorSubcoreMesh(
    core_axis_name="c", subcore_axis_name="s", num_cores=NC, num_subcores=NS
)


@functools.partial(
    pl.kernel,
    out_type=jax.ShapeDtypeStruct((N_EDGES, EDGE_EMB), jnp.float32),
    mesh=_SC_MESH,
    scratch_types=[
        pltpu.VMEM((N_EDGE_TYPE, EDGE_EMB), jnp.float32),
        pltpu.VMEM((EDGE_EMB, EDGE_EMB), jnp.float32),
        pltpu.VMEM((EDGE_EMB,), jnp.float32),
        pltpu.VMEM((N_EDGE_TYPE, EDGE_EMB), jnp.float32),
        pltpu.VMEM((EDGES_PER_W,), jnp.int32),
        pltpu.VMEM((CHUNK, EDGE_EMB), jnp.float32),
        pltpu.VMEM((CHUNK, EDGE_EMB), jnp.float32),
        pltpu.SemaphoreType.DMA,
        pltpu.SemaphoreType.DMA,
        pltpu.SemaphoreType.DMA,
    ],
    compiler_params=pltpu.CompilerParams(
        use_tc_tiling_on_sc=False, needs_layout_passes=False
    ),
)
def _edge_path_sc(
    idx_hbm, emb_hbm, w2t_hbm, b2_hbm, out_hbm,
    emb_v, w2t_v, b2_v, table_v, idx_v, rows0, rows1, sem_idx, sem_o0, sem_o1,
):
    wid = lax.axis_index("s") * NC + lax.axis_index("c")
    base_w = wid * EDGES_PER_W

    idx_dma = pltpu.async_copy(
        idx_hbm.at[pl.ds(base_w, EDGES_PER_W)], idx_v, sem_idx
    )

    pltpu.sync_copy(emb_hbm, emb_v)
    pltpu.sync_copy(w2t_hbm, w2t_v)
    pltpu.sync_copy(b2_hbm, b2_v)
    for i in range(N_EDGE_TYPE):
        e_row = emb_v[i, :]
        acc = b2_v[...]
        for k in range(EDGE_EMB):
            acc = acc + e_row[k] * w2t_v[k, :]
        table_v[i, :] = acc

    idx_dma.wait()

    iota16 = lax.iota(jnp.int32, LANES)
    cols = [jnp.full((LANES,), j, dtype=jnp.int32) for j in range(EDGE_EMB)]
    rows_bufs = (rows0, rows1)
    sems = (sem_o0, sem_o1)
    def compute_chunk(c, rb):
        ib_base = c * CHUNK

        @plsc.parallel_loop(0, CHUNK // LANES, unroll=4)
        def g_body(g):
            idxv = idx_v[pl.ds(ib_base + g * LANES, LANES)]
            row_pos = g * LANES + iota16
            for j in range(EDGE_EMB):
                vals = plsc.load_gather(table_v, [idxv, cols[j]])
                plsc.store_scatter(rb, [row_pos, cols[j]], vals)

    def drain(b):
        pltpu.make_async_copy(
            rows_bufs[b], out_hbm.at[pl.ds(base_w, CHUNK)], sems[b]
        ).wait()

    def pair_body(c2, carry):
        for b in range(2):
            c = 2 * c2 + b

            @pl.when(c2 > 0)
            def _():
                drain(b)

            compute_chunk(c, rows_bufs[b])
            pltpu.async_copy(
                rows_bufs[b],
                out_hbm.at[pl.ds(base_w + c * CHUNK, CHUNK)],
                sems[b],
            )
        return carry

    lax.fori_loop(0, N_CHUNKS // 2, pair_body, 0)

    drain(0)
    compute_chunk(N_CHUNKS - 1, rows0)
    pltpu.async_copy(
        rows0, out_hbm.at[pl.ds(base_w + (N_CHUNKS - 1) * CHUNK, CHUNK)], sem_o0
    )
    drain(0)
    drain(1)


NODE_BLK = 2000


def _node_mlp_body(x_ref, w_ref, b_ref, o_ref):
    o_ref[...] = (
        jnp.dot(x_ref[...], w_ref[...], preferred_element_type=jnp.float32)
        + b_ref[...]
    )


def _node_mlp(x, w1t, b1):
    grid = N_NODES // NODE_BLK
    return pl.pallas_call(
        _node_mlp_body,
        grid=(grid,),
        in_specs=[
            pl.BlockSpec((NODE_BLK, NODE_FEAT), lambda i: (i, 0)),
            pl.BlockSpec((NODE_FEAT, NODE_EMB), lambda i: (0, 0)),
            pl.BlockSpec((1, NODE_EMB), lambda i: (0, 0)),
        ],
        out_specs=pl.BlockSpec((NODE_BLK, NODE_EMB), lambda i: (i, 0)),
        out_shape=jax.ShapeDtypeStruct((N_NODES, NODE_EMB), jnp.float32),
    )(x, w1t, b1)


def kernel(raw_node_features, raw_edge_features, W1, b1, emb_table, W2, b2):
    edge_outputs = _edge_path_sc(raw_edge_features, emb_table, W2.T, b2)
    node_outputs = _node_mlp(raw_node_features, W1.T, b1.reshape(1, NODE_EMB))
    return (node_outputs, edge_outputs)


# speedup vs baseline: 5.0440x; 1.1213x over previous
import functools

import jax
import jax.numpy as jnp
from jax import lax
from jax.experimental import pallas as pl
from jax.experimental.pallas import tpu as pltpu
from jax.experimental.pallas import tpu_sc as plsc

N_NODES = 100000
N_EDGES = 1600000
NODE_FEAT = 128
NODE_EMB = 128
EDGE_EMB = 16
N_EDGE_TYPE = 16

NC = 2
NS = 16
NW = NC * NS
LANES = 16

EDGES_PER_W = N_EDGES // NW
CHUNK = 2000
N_CHUNKS = EDGES_PER_W // CHUNK

OUT_COLS = 128
OUT_ROWS = N_EDGES * EDGE_EMB // OUT_COLS
ROWS_PER_W = OUT_ROWS // NW
ROWS_PER_CHUNK = CHUNK * EDGE_EMB // OUT_COLS

_SC_MESH = plsc.VectorSubcoreMesh(
    core_axis_name="c", subcore_axis_name="s", num_cores=NC, num_subcores=NS
)


@functools.partial(
    pl.kernel,
    out_type=jax.ShapeDtypeStruct((OUT_ROWS, OUT_COLS), jnp.float32),
    mesh=_SC_MESH,
    scratch_types=[
        pltpu.VMEM((N_EDGE_TYPE, EDGE_EMB), jnp.float32),
        pltpu.VMEM((EDGE_EMB, EDGE_EMB), jnp.float32),
        pltpu.VMEM((EDGE_EMB,), jnp.float32),
        pltpu.VMEM((N_EDGE_TYPE, EDGE_EMB), jnp.float32),
        pltpu.VMEM((EDGES_PER_W,), jnp.int32),
        pltpu.VMEM((ROWS_PER_CHUNK, OUT_COLS), jnp.float32),
        pltpu.VMEM((ROWS_PER_CHUNK, OUT_COLS), jnp.float32),
        pltpu.SemaphoreType.DMA,
        pltpu.SemaphoreType.DMA,
        pltpu.SemaphoreType.DMA,
    ],
    compiler_params=pltpu.CompilerParams(
        use_tc_tiling_on_sc=False,
        needs_layout_passes=False,
        disable_bounds_checks=True,
    ),
)
def _edge_path_sc(
    idx_hbm, emb_hbm, w2t_hbm, b2_hbm, out_hbm,
    emb_v, w2t_v, b2_v, table_v, idx_v, rows0, rows1, sem_idx, sem_o0, sem_o1,
):
    wid = lax.axis_index("s") * NC + lax.axis_index("c")
    idx_base_w = wid * EDGES_PER_W
    row_base_w = wid * ROWS_PER_W

    idx_dma = pltpu.async_copy(
        idx_hbm.at[pl.ds(idx_base_w, EDGES_PER_W)], idx_v, sem_idx
    )

    pltpu.sync_copy(emb_hbm, emb_v)
    pltpu.sync_copy(w2t_hbm, w2t_v)
    pltpu.sync_copy(b2_hbm, b2_v)
    for i in range(N_EDGE_TYPE):
        e_row = emb_v[i, :]
        acc = b2_v[...]
        for k in range(EDGE_EMB):
            acc = acc + e_row[k] * w2t_v[k, :]
        table_v[i, :] = acc

    idx_dma.wait()

    iota16 = lax.iota(jnp.int32, LANES)
    cols = []
    row_offs = []
    for j in range(EDGE_EMB):
        flat = iota16 * EDGE_EMB + j
        cols.append(lax.bitwise_and(flat, OUT_COLS - 1))
        row_offs.append(lax.shift_right_logical(flat, 7))
    colsel = [jnp.full((LANES,), j, dtype=jnp.int32) for j in range(EDGE_EMB)]
    rows_bufs = (rows0, rows1)
    sems = (sem_o0, sem_o1)

    def compute_chunk(c, rb):
        ib_base = c * CHUNK

        @plsc.parallel_loop(0, CHUNK // LANES, unroll=4)
        def g_body(g):
            idxv = idx_v[pl.ds(ib_base + g * LANES, LANES)]
            grow = g * 2
            for j in range(EDGE_EMB):
                vals = plsc.load_gather(table_v, [idxv, colsel[j]])
                plsc.store_scatter(rb, [grow + row_offs[j], cols[j]], vals)

    def drain(b):
        pltpu.make_async_copy(
            rows_bufs[b], out_hbm.at[pl.ds(row_base_w, ROWS_PER_CHUNK)], sems[b]
        ).wait()

    def pair_body(c2, carry):
        for b in range(2):
            c = 2 * c2 + b

            @pl.when(c2 > 0)
            def _():
                drain(b)

            compute_chunk(c, rows_bufs[b])
            pltpu.async_copy(
                rows_bufs[b],
                out_hbm.at[pl.ds(row_base_w + c * ROWS_PER_CHUNK, ROWS_PER_CHUNK)],
                sems[b],
            )
        return carry

    lax.fori_loop(0, N_CHUNKS // 2, pair_body, 0)

    drain(0)
    compute_chunk(N_CHUNKS - 1, rows0)
    pltpu.async_copy(
        rows0,
        out_hbm.at[
            pl.ds(row_base_w + (N_CHUNKS - 1) * ROWS_PER_CHUNK, ROWS_PER_CHUNK)
        ],
        sem_o0,
    )
    drain(0)
    drain(1)


NODE_BLK = 2000


def _node_mlp_body(x_ref, w_ref, b_ref, o_ref):
    o_ref[...] = (
        jnp.dot(x_ref[...], w_ref[...], preferred_element_type=jnp.float32)
        + b_ref[...]
    )


def _node_mlp(x, w1t, b1):
    grid = N_NODES // NODE_BLK
    return pl.pallas_call(
        _node_mlp_body,
        grid=(grid,),
        in_specs=[
            pl.BlockSpec((NODE_BLK, NODE_FEAT), lambda i: (i, 0)),
            pl.BlockSpec((NODE_FEAT, NODE_EMB), lambda i: (0, 0)),
            pl.BlockSpec((1, NODE_EMB), lambda i: (0, 0)),
        ],
        out_specs=pl.BlockSpec((NODE_BLK, NODE_EMB), lambda i: (i, 0)),
        out_shape=jax.ShapeDtypeStruct((N_NODES, NODE_EMB), jnp.float32),
    )(x, w1t, b1)


def kernel(raw_node_features, raw_edge_features, W1, b1, emb_table, W2, b2):
    edge_flat = _edge_path_sc(raw_edge_features, emb_table, W2.T, b2)
    edge_outputs = edge_flat.reshape(N_EDGES, EDGE_EMB)
    node_outputs = _node_mlp(raw_node_features, W1.T, b1.reshape(1, NODE_EMB))
    return (node_outputs, edge_outputs)


# speedup vs baseline: 22.5706x; 4.4747x over previous
import functools

import jax
import jax.numpy as jnp
from jax import lax
from jax.experimental import pallas as pl
from jax.experimental.pallas import tpu as pltpu
from jax.experimental.pallas import tpu_sc as plsc

N_NODES = 100000
N_EDGES = 1600000
NODE_FEAT = 128
NODE_EMB = 128
EDGE_EMB = 16
N_EDGE_TYPE = 16

NC = 2
NS = 16
NW = NC * NS
LANES = 16

ETILE = 128
N_ETILES = N_EDGES // ETILE
JT = 2
JIN = 8
PART = N_ETILES * JIN * ETILE

TILES_PER_CHUNK = 20
CHUNK = TILES_PER_CHUNK * ETILE
N_CHUNKS_TOTAL = N_ETILES // TILES_PER_CHUNK
CHUNK_PART = TILES_PER_CHUNK * JIN * ETILE
CHUNK_FLOATS = JT * CHUNK_PART
N_FULL = N_CHUNKS_TOTAL - (N_CHUNKS_TOTAL // NW) * NW

_SC_MESH = plsc.VectorSubcoreMesh(
    core_axis_name="c", subcore_axis_name="s", num_cores=NC, num_subcores=NS
)


@functools.partial(
    pl.kernel,
    out_type=jax.ShapeDtypeStruct((N_EDGES * EDGE_EMB,), jnp.float32),
    mesh=_SC_MESH,
    scratch_types=[
        pltpu.VMEM((N_EDGE_TYPE, EDGE_EMB), jnp.float32),
        pltpu.VMEM((EDGE_EMB, EDGE_EMB), jnp.float32),
        pltpu.VMEM((EDGE_EMB,), jnp.float32),
        pltpu.VMEM((N_EDGE_TYPE, EDGE_EMB), jnp.float32),
        pltpu.VMEM((CHUNK,), jnp.int32),
        pltpu.VMEM((CHUNK,), jnp.int32),
        pltpu.VMEM((CHUNK_FLOATS,), jnp.float32),
        pltpu.VMEM((CHUNK_FLOATS,), jnp.float32),
        pltpu.SemaphoreType.DMA,
        pltpu.SemaphoreType.DMA,
        pltpu.SemaphoreType.DMA,
        pltpu.SemaphoreType.DMA,
    ],
    compiler_params=pltpu.CompilerParams(
        use_tc_tiling_on_sc=False,
        needs_layout_passes=False,
        disable_bounds_checks=True,
    ),
)
def _edge_path_sc(
    idx_hbm, emb_hbm, w2t_hbm, b2_hbm, out_hbm,
    emb_v, w2t_v, b2_v, table_v, idx0, idx1, rows0, rows1,
    sem_i0, sem_i1, sem_o0, sem_o1,
):
    wid = lax.axis_index("s") * NC + lax.axis_index("c")
    n_w = jnp.where(wid < N_FULL, 20, 19)

    idx_bufs = (idx0, idx1)
    rows_bufs = (rows0, rows1)
    sems_i = (sem_i0, sem_i1)
    sems_o = (sem_o0, sem_o1)

    def chunk_id(i):
        return wid + NW * i

    def issue_idx(i, b):
        pltpu.async_copy(
            idx_hbm.at[pl.ds(chunk_id(i) * CHUNK, CHUNK)], idx_bufs[b], sems_i[b]
        )

    def wait_idx(b):
        pltpu.make_async_copy(
            idx_hbm.at[pl.ds(0, CHUNK)], idx_bufs[b], sems_i[b]
        ).wait()

    def issue_out(i, b):
        k = chunk_id(i)
        rb = rows_bufs[b]
        pltpu.async_copy(
            rb.at[pl.ds(0, CHUNK_PART)],
            out_hbm.at[pl.ds(k * CHUNK_PART, CHUNK_PART)],
            sems_o[b],
        )
        pltpu.async_copy(
            rb.at[pl.ds(CHUNK_PART, CHUNK_PART)],
            out_hbm.at[pl.ds(PART + k * CHUNK_PART, CHUNK_PART)],
            sems_o[b],
        )

    def drain_out(b):
        pltpu.make_async_copy(
            rows_bufs[b], out_hbm.at[pl.ds(0, CHUNK_FLOATS)], sems_o[b]
        ).wait()

    issue_idx(0, 0)
    issue_idx(1, 1)

    pltpu.sync_copy(emb_hbm, emb_v)
    pltpu.sync_copy(w2t_hbm, w2t_v)
    pltpu.sync_copy(b2_hbm, b2_v)
    for i in range(N_EDGE_TYPE):
        e_row = emb_v[i, :]
        acc = b2_v[...]
        for k in range(EDGE_EMB):
            acc = acc + e_row[k] * w2t_v[k, :]
        table_v[i, :] = acc

    colsel = [jnp.full((LANES,), j, dtype=jnp.int32) for j in range(EDGE_EMB)]
    joff = [(j // JIN) * CHUNK_PART + (j % JIN) * ETILE for j in range(EDGE_EMB)]

    def compute_chunk(b):
        ib = idx_bufs[b]
        rb = rows_bufs[b]

        @plsc.parallel_loop(0, CHUNK // LANES, unroll=4)
        def g_body(g):
            idxv = ib[pl.ds(g * LANES, LANES)]
            base_g = (g >> 3) * (JIN * ETILE) + (g & 7) * LANES
            for j in range(EDGE_EMB):
                vals = plsc.load_gather(table_v, [idxv, colsel[j]])
                rb[pl.ds(base_g + joff[j], LANES)] = vals

    def process(i, b):
        wait_idx(b)

        @pl.when(i > 1)
        def _():
            drain_out(b)

        compute_chunk(b)
        issue_out(i, b)

        @pl.when(i + 2 < n_w)
        def _():
            issue_idx(i + 2, b)

    def pair_body(i2, carry):
        for b in range(2):
            process(2 * i2 + b, b)
        return carry

    lax.fori_loop(0, n_w >> 1, pair_body, 0)

    @pl.when((n_w & 1) == 1)
    def _():
        process(n_w - 1, 0)

    drain_out(0)
    drain_out(1)


NODE_BLK = 2000


def _node_mlp_body(x_ref, w_ref, b_ref, o_ref):
    o_ref[...] = (
        jnp.dot(x_ref[...], w_ref[...], preferred_element_type=jnp.float32)
        + b_ref[...]
    )


def _node_mlp(x, w1t, b1):
    grid = N_NODES // NODE_BLK
    return pl.pallas_call(
        _node_mlp_body,
        grid=(grid,),
        in_specs=[
            pl.BlockSpec((NODE_BLK, NODE_FEAT), lambda i: (i, 0)),
            pl.BlockSpec((NODE_FEAT, NODE_EMB), lambda i: (0, 0)),
            pl.BlockSpec((1, NODE_EMB), lambda i: (0, 0)),
        ],
        out_specs=pl.BlockSpec((NODE_BLK, NODE_EMB), lambda i: (i, 0)),
        out_shape=jax.ShapeDtypeStruct((N_NODES, NODE_EMB), jnp.float32),
    )(x, w1t, b1)


def kernel(raw_node_features, raw_edge_features, W1, b1, emb_table, W2, b2):
    edge_flat = _edge_path_sc(raw_edge_features, emb_table, W2.T, b2)
    edge_outputs = (
        edge_flat.reshape(JT, N_ETILES, JIN, ETILE)
        .transpose(1, 3, 0, 2)
        .reshape(N_EDGES, EDGE_EMB)
    )
    node_outputs = _node_mlp(raw_node_features, W1.T, b1.reshape(1, NODE_EMB))
    return (node_outputs, edge_outputs)


# speedup vs baseline: 46.8833x; 2.0772x over previous
import functools

import jax
import jax.numpy as jnp
from jax import lax
from jax.experimental import pallas as pl
from jax.experimental.pallas import tpu as pltpu
from jax.experimental.pallas import tpu_sc as plsc

N_NODES = 100000
N_EDGES = 1600000
NODE_FEAT = 128
NODE_EMB = 128
EDGE_EMB = 16
N_EDGE_TYPE = 16

NC = 2
NS = 16
NW = NC * NS
LANES = 16

ETILE = 128
N_ETILES = N_EDGES // ETILE
JT = 2
JIN = 8
PART = N_ETILES * JIN * ETILE

TILES_PER_CHUNK = 20
CHUNK = TILES_PER_CHUNK * ETILE
N_CHUNKS_TOTAL = N_ETILES // TILES_PER_CHUNK
CHUNK_PART = TILES_PER_CHUNK * JIN * ETILE
CHUNK_FLOATS = JT * CHUNK_PART
N_FULL = N_CHUNKS_TOTAL - (N_CHUNKS_TOTAL // NW) * NW

_SC_MESH = plsc.VectorSubcoreMesh(
    core_axis_name="c", subcore_axis_name="s", num_cores=NC, num_subcores=NS
)


@functools.partial(
    pl.kernel,
    out_type=jax.ShapeDtypeStruct((N_EDGES * EDGE_EMB,), jnp.float32),
    mesh=_SC_MESH,
    scratch_types=[
        pltpu.VMEM((N_EDGE_TYPE, EDGE_EMB), jnp.float32),
        pltpu.VMEM((EDGE_EMB, EDGE_EMB), jnp.float32),
        pltpu.VMEM((EDGE_EMB,), jnp.float32),
        pltpu.VMEM((N_EDGE_TYPE, EDGE_EMB), jnp.float32),
        pltpu.VMEM((CHUNK,), jnp.int32),
        pltpu.VMEM((CHUNK,), jnp.int32),
        pltpu.VMEM((CHUNK_FLOATS,), jnp.float32),
        pltpu.VMEM((CHUNK_FLOATS,), jnp.float32),
        pltpu.SemaphoreType.DMA,
        pltpu.SemaphoreType.DMA,
        pltpu.SemaphoreType.DMA,
        pltpu.SemaphoreType.DMA,
    ],
    compiler_params=pltpu.CompilerParams(
        use_tc_tiling_on_sc=False,
        needs_layout_passes=False,
        disable_bounds_checks=True,
    ),
)
def _edge_path_sc(
    idx_hbm, embt_hbm, w2_hbm, b2_hbm, out_hbm,
    embt_v, w2_v, b2_v, table_v, idx0, idx1, rows0, rows1,
    sem_i0, sem_i1, sem_o0, sem_o1,
):
    wid = lax.axis_index("s") * NC + lax.axis_index("c")
    n_w = jnp.where(wid < N_FULL, 20, 19)

    idx_bufs = (idx0, idx1)
    rows_bufs = (rows0, rows1)
    sems_i = (sem_i0, sem_i1)
    sems_o = (sem_o0, sem_o1)

    def chunk_id(i):
        return wid + NW * i

    def issue_idx(i, b):
        pltpu.async_copy(
            idx_hbm.at[pl.ds(chunk_id(i) * CHUNK, CHUNK)], idx_bufs[b], sems_i[b]
        )

    def wait_idx(b):
        pltpu.make_async_copy(
            idx_hbm.at[pl.ds(0, CHUNK)], idx_bufs[b], sems_i[b]
        ).wait()

    def issue_out(i, b):
        k = chunk_id(i)
        rb = rows_bufs[b]
        pltpu.async_copy(
            rb.at[pl.ds(0, CHUNK_PART)],
            out_hbm.at[pl.ds(k * CHUNK_PART, CHUNK_PART)],
            sems_o[b],
        )
        pltpu.async_copy(
            rb.at[pl.ds(CHUNK_PART, CHUNK_PART)],
            out_hbm.at[pl.ds(PART + k * CHUNK_PART, CHUNK_PART)],
            sems_o[b],
        )

    def drain_out(b):
        pltpu.make_async_copy(
            rows_bufs[b], out_hbm.at[pl.ds(0, CHUNK_FLOATS)], sems_o[b]
        ).wait()

    issue_idx(0, 0)
    issue_idx(1, 1)

    pltpu.sync_copy(embt_hbm, embt_v)
    pltpu.sync_copy(w2_hbm, w2_v)
    pltpu.sync_copy(b2_hbm, b2_v)
    b2_row = b2_v[...]
    for j in range(N_EDGE_TYPE):
        w_row = w2_v[j, :]
        acc = jnp.full((LANES,), 0.0, dtype=jnp.float32) + b2_row[j]
        for k in range(EDGE_EMB):
            acc = acc + w_row[k] * embt_v[k, :]
        table_v[j, :] = acc

    colsel = [jnp.full((LANES,), j, dtype=jnp.int32) for j in range(EDGE_EMB)]
    joff = [(j // JIN) * CHUNK_PART + (j % JIN) * ETILE for j in range(EDGE_EMB)]

    def compute_chunk(b):
        ib = idx_bufs[b]
        rb = rows_bufs[b]

        @plsc.parallel_loop(0, CHUNK // LANES, unroll=4)
        def g_body(g):
            idxv = ib[pl.ds(g * LANES, LANES)]
            base_g = (g >> 3) * (JIN * ETILE) + (g & 7) * LANES
            for j in range(EDGE_EMB):
                vals = plsc.load_gather(table_v, [colsel[j], idxv])
                rb[pl.ds(base_g + joff[j], LANES)] = vals

    def process(i, b):
        wait_idx(b)

        @pl.when(i > 1)
        def _():
            drain_out(b)

        compute_chunk(b)
        issue_out(i, b)

        @pl.when(i + 2 < n_w)
        def _():
            issue_idx(i + 2, b)

    def pair_body(i2, carry):
        for b in range(2):
            process(2 * i2 + b, b)
        return carry

    lax.fori_loop(0, n_w >> 1, pair_body, 0)

    @pl.when((n_w & 1) == 1)
    def _():
        process(n_w - 1, 0)

    drain_out(0)
    drain_out(1)


NODE_BLK = 2000


def _node_mlp_body(x_ref, w_ref, b_ref, o_ref):
    o_ref[...] = (
        jnp.dot(x_ref[...], w_ref[...], preferred_element_type=jnp.float32)
        + b_ref[...]
    )


def _node_mlp(x, w1t, b1):
    grid = N_NODES // NODE_BLK
    return pl.pallas_call(
        _node_mlp_body,
        grid=(grid,),
        in_specs=[
            pl.BlockSpec((NODE_BLK, NODE_FEAT), lambda i: (i, 0)),
            pl.BlockSpec((NODE_FEAT, NODE_EMB), lambda i: (0, 0)),
            pl.BlockSpec((1, NODE_EMB), lambda i: (0, 0)),
        ],
        out_specs=pl.BlockSpec((NODE_BLK, NODE_EMB), lambda i: (i, 0)),
        out_shape=jax.ShapeDtypeStruct((N_NODES, NODE_EMB), jnp.float32),
    )(x, w1t, b1)


def kernel(raw_node_features, raw_edge_features, W1, b1, emb_table, W2, b2):
    edge_flat = _edge_path_sc(raw_edge_features, emb_table.T, W2, b2)
    edge_outputs = (
        edge_flat.reshape(JT, N_ETILES, JIN, ETILE)
        .transpose(1, 3, 0, 2)
        .reshape(N_EDGES, EDGE_EMB)
    )
    node_outputs = _node_mlp(raw_node_features, W1.T, b1.reshape(1, NODE_EMB))
    return (node_outputs, edge_outputs)


# speedup vs baseline: 47.5479x; 1.0142x over previous
import functools

import jax
import jax.numpy as jnp
from jax import lax
from jax.experimental import pallas as pl
from jax.experimental.pallas import tpu as pltpu
from jax.experimental.pallas import tpu_sc as plsc

N_NODES = 100000
N_EDGES = 1600000
NODE_FEAT = 128
NODE_EMB = 128
EDGE_EMB = 16
N_EDGE_TYPE = 16

NC = 2
NS = 16
NW = NC * NS
LANES = 16

ETILE = 128
N_ETILES = N_EDGES // ETILE
JT = 2
JIN = 8
PART = N_ETILES * JIN * ETILE

TILES_PER_CHUNK = 20
CHUNK = TILES_PER_CHUNK * ETILE
N_CHUNKS_TOTAL = N_ETILES // TILES_PER_CHUNK
CHUNK_PART = TILES_PER_CHUNK * JIN * ETILE
CHUNK_FLOATS = JT * CHUNK_PART
N_FULL = N_CHUNKS_TOTAL - (N_CHUNKS_TOTAL // NW) * NW

_SC_MESH = plsc.VectorSubcoreMesh(
    core_axis_name="c", subcore_axis_name="s", num_cores=NC, num_subcores=NS
)


@functools.partial(
    pl.kernel,
    out_type=jax.ShapeDtypeStruct((N_EDGES * EDGE_EMB,), jnp.float32),
    mesh=_SC_MESH,
    scratch_types=[
        pltpu.VMEM((N_EDGE_TYPE, EDGE_EMB), jnp.float32),
        pltpu.VMEM((EDGE_EMB, EDGE_EMB), jnp.float32),
        pltpu.VMEM((EDGE_EMB,), jnp.float32),
        pltpu.VMEM((N_EDGE_TYPE, EDGE_EMB), jnp.float32),
        pltpu.VMEM((CHUNK,), jnp.int32),
        pltpu.VMEM((CHUNK,), jnp.int32),
        pltpu.VMEM((CHUNK_FLOATS,), jnp.float32),
        pltpu.VMEM((CHUNK_FLOATS,), jnp.float32),
        pltpu.SemaphoreType.DMA,
        pltpu.SemaphoreType.DMA,
        pltpu.SemaphoreType.DMA,
        pltpu.SemaphoreType.DMA,
    ],
    compiler_params=pltpu.CompilerParams(
        use_tc_tiling_on_sc=False,
        needs_layout_passes=False,
        disable_bounds_checks=True,
    ),
)
def _edge_path_sc(
    idx_hbm, embt_hbm, w2_hbm, b2_hbm, out_hbm,
    embt_v, w2_v, b2_v, table_v, idx0, idx1, rows0, rows1,
    sem_i0, sem_i1, sem_o0, sem_o1,
):
    wid = lax.axis_index("s") * NC + lax.axis_index("c")
    n_w = jnp.where(wid < N_FULL, 20, 19)

    idx_bufs = (idx0, idx1)
    rows_bufs = (rows0, rows1)
    sems_i = (sem_i0, sem_i1)
    sems_o = (sem_o0, sem_o1)

    def chunk_id(i):
        return wid + NW * i

    def issue_idx(i, b):
        pltpu.async_copy(
            idx_hbm.at[pl.ds(chunk_id(i) * CHUNK, CHUNK)], idx_bufs[b], sems_i[b]
        )

    def wait_idx(b):
        pltpu.make_async_copy(
            idx_hbm.at[pl.ds(0, CHUNK)], idx_bufs[b], sems_i[b]
        ).wait()

    def issue_out(i, b):
        k = chunk_id(i)
        rb = rows_bufs[b]
        pltpu.async_copy(
            rb.at[pl.ds(0, CHUNK_PART)],
            out_hbm.at[pl.ds(k * CHUNK_PART, CHUNK_PART)],
            sems_o[b],
        )
        pltpu.async_copy(
            rb.at[pl.ds(CHUNK_PART, CHUNK_PART)],
            out_hbm.at[pl.ds(PART + k * CHUNK_PART, CHUNK_PART)],
            sems_o[b],
        )

    def drain_out(b):
        pltpu.make_async_copy(
            rows_bufs[b], out_hbm.at[pl.ds(0, CHUNK_FLOATS)], sems_o[b]
        ).wait()

    issue_idx(0, 0)
    issue_idx(1, 1)

    pltpu.sync_copy(embt_hbm, embt_v)
    pltpu.sync_copy(w2_hbm, w2_v)
    pltpu.sync_copy(b2_hbm, b2_v)
    b2_row = b2_v[...]
    for j in range(N_EDGE_TYPE):
        w_row = w2_v[j, :]
        acc = jnp.full((LANES,), 0.0, dtype=jnp.float32) + b2_row[j]
        for k in range(EDGE_EMB):
            acc = acc + w_row[k] * embt_v[k, :]
        table_v[j, :] = acc

    colsel = [jnp.full((LANES,), j, dtype=jnp.int32) for j in range(EDGE_EMB)]
    joff = [(j // JIN) * CHUNK_PART + (j % JIN) * ETILE for j in range(EDGE_EMB)]

    def compute_chunk(b):
        ib = idx_bufs[b]
        rb = rows_bufs[b]

        @plsc.parallel_loop(0, CHUNK // LANES, unroll=4)
        def g_body(g):
            idxv = ib[pl.ds(g * LANES, LANES)]
            base_g = (g >> 3) * (JIN * ETILE) + (g & 7) * LANES
            for j in range(EDGE_EMB):
                vals = plsc.load_gather(table_v, [colsel[j], idxv])
                rb[pl.ds(base_g + joff[j], LANES)] = vals

    def process(i, b):
        wait_idx(b)

        @pl.when(i > 1)
        def _():
            drain_out(b)

        compute_chunk(b)
        issue_out(i, b)

        @pl.when(i + 2 < n_w)
        def _():
            issue_idx(i + 2, b)

    def pair_body(i2, carry):
        for b in range(2):
            process(2 * i2 + b, b)
        return carry

    lax.fori_loop(0, n_w >> 1, pair_body, 0)

    @pl.when((n_w & 1) == 1)
    def _():
        process(n_w - 1, 0)

    drain_out(0)
    drain_out(1)


NODE_BLK = 5000


def _node_mlp_body(x_ref, w_ref, b_ref, o_ref):
    o_ref[...] = (
        jnp.dot(x_ref[...], w_ref[...], preferred_element_type=jnp.float32)
        + b_ref[...]
    )


def _node_mlp(x, w1t, b1):
    grid = N_NODES // NODE_BLK
    return pl.pallas_call(
        _node_mlp_body,
        grid=(grid,),
        in_specs=[
            pl.BlockSpec((NODE_BLK, NODE_FEAT), lambda i: (i, 0)),
            pl.BlockSpec((NODE_FEAT, NODE_EMB), lambda i: (0, 0)),
            pl.BlockSpec((1, NODE_EMB), lambda i: (0, 0)),
        ],
        out_specs=pl.BlockSpec((NODE_BLK, NODE_EMB), lambda i: (i, 0)),
        out_shape=jax.ShapeDtypeStruct((N_NODES, NODE_EMB), jnp.float32),
    )(x, w1t, b1)


def kernel(raw_node_features, raw_edge_features, W1, b1, emb_table, W2, b2):
    edge_flat = _edge_path_sc(raw_edge_features, emb_table.T, W2, b2)
    edge_outputs = (
        edge_flat.reshape(JT, N_ETILES, JIN, ETILE)
        .transpose(1, 3, 0, 2)
        .reshape(N_EDGES, EDGE_EMB)
    )
    node_outputs = _node_mlp(raw_node_features, W1.T, b1.reshape(1, NODE_EMB))
    return (node_outputs, edge_outputs)
